# baseline (speedup 1.0000x reference)
import jax
import numpy as np
import jax.numpy as jnp
from jax import lax
from jax.experimental import pallas as pl
from jax.experimental.pallas import tpu as pltpu
from jax.experimental.pallas import tpu_sc as plsc

TOKENS = 8192
D_MODEL = 768
FACTORS = 32768
TOPK = 64

TB = 512
FB = 1024


BLK = 128
NBLK = FACTORS // BLK
NF = FACTORS // FB


def _matmul_body(x_ref, w_ref, b_ref, out_ref, g_ref):
    xc = x_ref[...] - b_ref[...]
    p = jnp.dot(xc, w_ref[...], preferred_element_type=jnp.float32)
    out_ref[...] = p
    g = jnp.max(p.reshape(TB, FB // BLK, BLK), axis=2)
    g_ref[...] = g.reshape(1, TB, FB // BLK)


def _pre_acts(x, W_enc, b_pre):
    return pl.pallas_call(
        _matmul_body,
        grid=(FACTORS // FB, TOKENS // TB),
        in_specs=[
            pl.BlockSpec((TB, D_MODEL), lambda f, t: (t, 0)),
            pl.BlockSpec((D_MODEL, FB), lambda f, t: (0, f)),
            pl.BlockSpec((1, D_MODEL), lambda f, t: (0, 0)),
        ],
        out_specs=[
            pl.BlockSpec((TB, FB), lambda f, t: (t, f)),
            pl.BlockSpec((1, TB, FB // BLK), lambda f, t: (f, t, 0)),
        ],
        out_shape=[
            jax.ShapeDtypeStruct((TOKENS, FACTORS), jnp.float32),
            jax.ShapeDtypeStruct((NF, TOKENS, FB // BLK), jnp.float32),
        ],
        compiler_params=pltpu.CompilerParams(
            dimension_semantics=("arbitrary", "arbitrary"),
            vmem_limit_bytes=100 * 1024 * 1024,
        ),
    )(x, W_enc, b_pre.reshape(1, D_MODEL))


KB = 2048


def _dec_body(p_ref, w_ref, b_ref, tau_ref, acts_ref, out_ref):
    k = pl.program_id(1)
    p = p_ref[...]
    a = jnp.where(p >= tau_ref[...], p, 0.0)
    acts_ref[...] = a

    @pl.when(k == 0)
    def _():
        out_ref[...] = jnp.broadcast_to(b_ref[...], out_ref.shape)

    out_ref[...] += jnp.dot(
        a, w_ref[...], preferred_element_type=jnp.float32
    )


def _decode(pre, W_dec, b_pre, tau):
    return pl.pallas_call(
        _dec_body,
        grid=(TOKENS // TB, FACTORS // KB),
        in_specs=[
            pl.BlockSpec((TB, KB), lambda t, k: (t, k)),
            pl.BlockSpec((KB, D_MODEL), lambda t, k: (k, 0)),
            pl.BlockSpec((1, D_MODEL), lambda t, k: (0, 0)),
            pl.BlockSpec((TB, 1), lambda t, k: (t, 0)),
        ],
        out_specs=[
            pl.BlockSpec((TB, KB), lambda t, k: (t, k)),
            pl.BlockSpec((TB, D_MODEL), lambda t, k: (t, 0)),
        ],
        out_shape=[
            jax.ShapeDtypeStruct((TOKENS, FACTORS), jnp.float32),
            jax.ShapeDtypeStruct((TOKENS, D_MODEL), jnp.float32),
        ],
        compiler_params=pltpu.CompilerParams(
            dimension_semantics=("arbitrary", "arbitrary"),
        ),
    )(pre, W_dec, b_pre.reshape(1, D_MODEL), tau)


SC_CORES = 2
SC_SUBCORES = 16
LANES = 16
NW = SC_CORES * SC_SUBCORES
ROWS_PER_W = TOKENS // NW
NVEC = FACTORS // LANES
GROUPS = 256
GVREGS = GROUPS // LANES
VPG = NVEC // GROUPS * LANES
CAP = 512
BLCAP = 512

_SIGN = np.uint32(0x80000000)
_MANT = np.uint32(0x7FFFFFFF)


def _mkey(v):
    b = lax.bitcast_convert_type(v, jnp.uint32)
    neg = (b >> 31) == 1
    return jnp.where(neg, ~b, b | _SIGN)


def _inv_key(u):
    is_pos = (u >> 31) == 1
    b = jnp.where(is_pos, u & _MANT, ~u)
    return lax.bitcast_convert_type(b, jnp.float32)


def _sc_body(pre, gmax, tau, row_a, row_b, g_a, g_b, blist_v,
             cidx_v, ckey_v, tau_buf, sem_a, sem_b, sem_ga, sem_gb):
    wid = lax.axis_index("s") * SC_CORES + lax.axis_index("c")
    r0 = wid * ROWS_PER_W
    lanes = lax.iota(jnp.int32, LANES)
    zero16f = jnp.zeros((LANES,), jnp.float32)
    zero16i = jnp.zeros((LANES,), jnp.int32)

    def _z_cidx(i, c):
        cidx_v[pl.ds(i * LANES, LANES)] = zero16i
        return c

    lax.fori_loop(0, CAP // LANES, _z_cidx, 0)

    def _z_blist(i, c):
        blist_v[pl.ds(i * LANES, LANES)] = zero16i
        return c

    lax.fori_loop(0, BLCAP // LANES, _z_blist, 0)

    NGV = NBLK // LANES

    def process(row_v, g_v, i):
        gks = [_mkey(g_v[pl.ds(b * LANES, LANES)]) for b in range(NGV)]

        def bs1(_, lohi):
            lo, hi = lohi
            mid = lo + ((hi - lo) >> 1) + np.uint32(1)
            cvec = zero16i
            for gv in gks:
                cvec = cvec + jnp.where(gv >= mid, 1, 0)
            c = jnp.sum(cvec)
            ok = c >= TOPK
            return (jnp.where(ok, mid, lo), jnp.where(ok, hi, mid - 1))

        tau_m, _ = lax.fori_loop(
            0, 32, bs1, (np.uint32(0), np.uint32(0xFFFFFFFF))
        )
        tau_f = _inv_key(tau_m)

        def blb(g8, bp):
            for k in range(8):
                g = g8 * 8 + k
                m = g_v[pl.ds(g * LANES, LANES)] >= tau_f
                ids = lanes + g * LANES
                plsc.store_compressed(blist_v.at[pl.ds(bp, LANES)], ids, mask=m)
                bp = jnp.minimum(
                    bp + plsc.all_reduce_population_count(m)[0], BLCAP - LANES
                )
            return bp

        bpos = lax.fori_loop(0, NGV // 8, blb, 0)

        def ext(bj, pos):
            b = blist_v[pl.ds(bj, LANES)][0]
            for k in range(BLK // LANES):
                off = b * BLK + k * LANES
                v = row_v[pl.ds(off, LANES)]
                mm = v >= tau_f
                iv = lanes + off
                plsc.store_compressed(cidx_v.at[pl.ds(pos, LANES)], iv, mask=mm)
                c = plsc.all_reduce_population_count(mm)[0]
                pos = jnp.minimum(pos + c, CAP - LANES)
            return pos

        pos = lax.fori_loop(0, bpos, ext, 0)
        vn = (pos + LANES - 1) // LANES

        def mk(j, c):
            iv = cidx_v[pl.ds(j * LANES, LANES)]
            vals = plsc.load_gather(row_v, [iv])
            valid = (lanes + j * LANES) < pos
            ckey_v[pl.ds(j * LANES, LANES)] = jnp.where(
                valid, _mkey(vals), np.uint32(0)
            )
            return c

        lax.fori_loop(0, vn, mk, 0)

        def bs2(_, lohi):
            lo, hi = lohi
            mid = lo + ((hi - lo) >> 1) + np.uint32(1)

            def cnt(j, acc):
                m = ckey_v[pl.ds(j * LANES, LANES)] >= mid
                return acc + jnp.where(m, 1, 0)

            c = jnp.sum(lax.fori_loop(0, vn, cnt, zero16i))
            ok = c >= TOPK
            return (jnp.where(ok, mid, lo), jnp.where(ok, hi, mid - 1))

        tau_s, _ = lax.fori_loop(
            0, 32, bs2, (np.uint32(0), np.uint32(0xFFFFFFFF))
        )
        tau_sf = _inv_key(tau_s)

        slot = (i >> 4) * LANES
        old = tau_buf[pl.ds(slot, LANES)]
        sel = lanes == (i & 15)
        tau_buf[pl.ds(slot, LANES)] = jnp.where(
            sel, jnp.full((LANES,), tau_sf), old
        )

    def _start(r, row_v, sem, g_v, gsem):
        pltpu.make_async_copy(pre.at[r], row_v, sem).start()
        pltpu.make_async_copy(gmax.at[r], g_v, gsem).start()

    def _wait(r, row_v, sem, g_v, gsem):
        pltpu.make_async_copy(pre.at[r], row_v, sem).wait()
        pltpu.make_async_copy(gmax.at[r], g_v, gsem).wait()

    _start(r0, row_a, sem_a, g_a, sem_ga)
    _start(r0 + 1, row_b, sem_b, g_b, sem_gb)

    def pair(i, c):
        ra = r0 + 2 * i
        _wait(ra, row_a, sem_a, g_a, sem_ga)
        process(row_a, g_a, 2 * i)
        lax.cond(
            2 * i + 2 < ROWS_PER_W,
            lambda: _start(ra + 2, row_a, sem_a, g_a, sem_ga),
            lambda: None,
        )
        rb = ra + 1
        _wait(rb, row_b, sem_b, g_b, sem_gb)
        process(row_b, g_b, 2 * i + 1)
        lax.cond(
            2 * i + 3 < ROWS_PER_W,
            lambda: _start(rb + 2, row_b, sem_b, g_b, sem_gb),
            lambda: None,
        )
        return c

    lax.fori_loop(0, ROWS_PER_W // 2, pair, 0)
    pltpu.sync_copy(tau_buf, tau.at[pl.ds(r0, ROWS_PER_W)])


def _sc_select(pre, gmax):
    mesh = plsc.VectorSubcoreMesh(
        core_axis_name="c", subcore_axis_name="s"
    )
    return pl.kernel(
        _sc_body,
        out_type=jax.ShapeDtypeStruct((TOKENS,), jnp.float32),
        mesh=mesh,
        compiler_params=pltpu.CompilerParams(needs_layout_passes=False),
        scratch_types=[
            pltpu.VMEM((FACTORS,), jnp.float32),
            pltpu.VMEM((FACTORS,), jnp.float32),
            pltpu.VMEM((NBLK,), jnp.float32),
            pltpu.VMEM((NBLK,), jnp.float32),
            pltpu.VMEM((BLCAP,), jnp.int32),
            pltpu.VMEM((CAP,), jnp.int32),
            pltpu.VMEM((CAP,), jnp.uint32),
            pltpu.VMEM((ROWS_PER_W,), jnp.float32),
            pltpu.SemaphoreType.DMA,
            pltpu.SemaphoreType.DMA,
            pltpu.SemaphoreType.DMA,
            pltpu.SemaphoreType.DMA,
        ],
    )(pre, gmax)


def kernel(x, W_enc, W_dec, b_pre):
    pre, g3 = _pre_acts(x, W_enc, b_pre)
    gmax = jnp.transpose(g3, (1, 0, 2)).reshape(TOKENS, NBLK)
    tau = _sc_select(pre, gmax)
    acts, x_hat = _decode(pre, W_dec, b_pre, tau.reshape(TOKENS, 1))
    return (x_hat, acts)


# speedup vs baseline: 1.3925x; 1.3925x over previous
import jax
import numpy as np
import jax.numpy as jnp
from jax import lax
from jax.experimental import pallas as pl
from jax.experimental.pallas import tpu as pltpu
from jax.experimental.pallas import tpu_sc as plsc

TOKENS = 8192
D_MODEL = 768
FACTORS = 32768
TOPK = 64

TB = 512
FB = 1024


BLK = 128
NBLK = FACTORS // BLK
NF = FACTORS // FB


def _matmul_body(x_ref, w_ref, b_ref, out_ref, g_ref):
    xc = x_ref[...] - b_ref[...]
    p = jnp.dot(xc, w_ref[...], preferred_element_type=jnp.float32)
    out_ref[...] = p
    g = jnp.max(p.reshape(TB, FB // BLK, BLK), axis=2)
    g_ref[...] = g.reshape(1, TB, FB // BLK)


def _pre_acts(x, W_enc, b_pre):
    return pl.pallas_call(
        _matmul_body,
        grid=(FACTORS // FB, TOKENS // TB),
        in_specs=[
            pl.BlockSpec((TB, D_MODEL), lambda f, t: (t, 0)),
            pl.BlockSpec((D_MODEL, FB), lambda f, t: (0, f)),
            pl.BlockSpec((1, D_MODEL), lambda f, t: (0, 0)),
        ],
        out_specs=[
            pl.BlockSpec((TB, FB), lambda f, t: (t, f)),
            pl.BlockSpec((1, TB, FB // BLK), lambda f, t: (f, t, 0)),
        ],
        out_shape=[
            jax.ShapeDtypeStruct((TOKENS, FACTORS), jnp.float32),
            jax.ShapeDtypeStruct((NF, TOKENS, FB // BLK), jnp.float32),
        ],
        compiler_params=pltpu.CompilerParams(
            dimension_semantics=("arbitrary", "arbitrary"),
            vmem_limit_bytes=100 * 1024 * 1024,
        ),
    )(x, W_enc, b_pre.reshape(1, D_MODEL))


KB = 4096


def _dec_body(a_ref, w_ref, b_ref, out_ref):
    k = pl.program_id(1)

    @pl.when(k == 0)
    def _():
        out_ref[...] = jnp.broadcast_to(b_ref[...], out_ref.shape)

    out_ref[...] += jnp.dot(
        a_ref[...], w_ref[...], preferred_element_type=jnp.float32
    )


def _decode(acts, W_dec, b_pre):
    return pl.pallas_call(
        _dec_body,
        grid=(TOKENS // TB, FACTORS // KB),
        in_specs=[
            pl.BlockSpec((TB, KB), lambda t, k: (t, k)),
            pl.BlockSpec((KB, D_MODEL), lambda t, k: (k, 0)),
            pl.BlockSpec((1, D_MODEL), lambda t, k: (0, 0)),
        ],
        out_specs=pl.BlockSpec((TB, D_MODEL), lambda t, k: (t, 0)),
        out_shape=jax.ShapeDtypeStruct((TOKENS, D_MODEL), jnp.float32),
        compiler_params=pltpu.CompilerParams(
            dimension_semantics=("arbitrary", "arbitrary"),
        ),
    )(acts, W_dec, b_pre.reshape(1, D_MODEL))


SC_CORES = 2
SC_SUBCORES = 16
LANES = 16
NW = SC_CORES * SC_SUBCORES
ROWS_PER_W = TOKENS // NW
NVEC = FACTORS // LANES
GROUPS = 256
GVREGS = GROUPS // LANES
VPG = NVEC // GROUPS * LANES
CAP = 512
BLCAP = 512

_SIGN = np.uint32(0x80000000)
_MANT = np.uint32(0x7FFFFFFF)


def _mkey(v):
    b = lax.bitcast_convert_type(v, jnp.uint32)
    neg = (b >> 31) == 1
    return jnp.where(neg, ~b, b | _SIGN)


def _inv_key(u):
    is_pos = (u >> 31) == 1
    b = jnp.where(is_pos, u & _MANT, ~u)
    return lax.bitcast_convert_type(b, jnp.float32)


def _sc_body(pre, gmax, acts, row_a, row_b, g_a, g_b, gk_v, blist_v,
             cidx_v, ckey_v, selidx_v, selidx_p, act_row,
             sem_a, sem_b, sem_ga, sem_gb, sem_o):
    wid = lax.axis_index("s") * SC_CORES + lax.axis_index("c")
    r0 = wid * ROWS_PER_W
    lanes = lax.iota(jnp.int32, LANES)
    zero16f = jnp.zeros((LANES,), jnp.float32)
    zero16i = jnp.zeros((LANES,), jnp.int32)

    def _z_act(i, c):
        act_row[pl.ds(i * LANES, LANES)] = zero16f
        return c

    lax.fori_loop(0, NVEC, _z_act, 0)

    def _z_cidx(i, c):
        cidx_v[pl.ds(i * LANES, LANES)] = zero16i
        return c

    lax.fori_loop(0, CAP // LANES, _z_cidx, 0)

    def _z_blist(i, c):
        blist_v[pl.ds(i * LANES, LANES)] = zero16i
        return c

    lax.fori_loop(0, BLCAP // LANES, _z_blist, 0)
    for j in range(TOPK // LANES):
        selidx_v[pl.ds(j * LANES, LANES)] = zero16i
        selidx_p[pl.ds(j * LANES, LANES)] = zero16i

    NGV = NBLK // LANES

    def process(row_v, g_v, r, first):
        gks = [_mkey(g_v[pl.ds(b * LANES, LANES)]) for b in range(NGV)]

        def bs1(_, lohi):
            lo, hi = lohi
            mid = lo + ((hi - lo) >> 1) + np.uint32(1)
            cvec = zero16i
            for gv in gks:
                cvec = cvec + jnp.where(gv >= mid, 1, 0)
            c = jnp.sum(cvec)
            ok = c >= TOPK
            return (jnp.where(ok, mid, lo), jnp.where(ok, hi, mid - 1))

        tau_m, _ = lax.fori_loop(
            0, 20, bs1, (np.uint32(0), np.uint32(0xFFFFFFFF))
        )
        tau_f = _inv_key(tau_m)

        def blb(g8, bp):
            for k in range(8):
                g = g8 * 8 + k
                m = g_v[pl.ds(g * LANES, LANES)] >= tau_f
                ids = lanes + g * LANES
                plsc.store_compressed(blist_v.at[pl.ds(bp, LANES)], ids, mask=m)
                bp = jnp.minimum(
                    bp + plsc.all_reduce_population_count(m)[0], BLCAP - LANES
                )
            return bp

        bpos = lax.fori_loop(0, NGV // 8, blb, 0)

        def ext(bj, pos):
            b = blist_v[pl.ds(bj, LANES)][0]
            mms = []
            cs = []
            for k in range(BLK // LANES):
                off = b * BLK + k * LANES
                mm = row_v[pl.ds(off, LANES)] >= tau_f
                mms.append((off, mm))
                cs.append(plsc.all_reduce_population_count(mm)[0])
            offs = [pos]
            for k in range(1, BLK // LANES):
                offs.append(jnp.minimum(offs[-1] + cs[k - 1], CAP - LANES))
            for k, (off, mm) in enumerate(mms):
                iv = lanes + off
                plsc.store_compressed(
                    cidx_v.at[pl.ds(offs[k], LANES)], iv, mask=mm
                )
            return jnp.minimum(offs[-1] + cs[-1], CAP - LANES)

        pos = lax.fori_loop(0, bpos, ext, 0)
        vn = (pos + LANES - 1) // LANES

        def mk(j, c):
            iv = cidx_v[pl.ds(j * LANES, LANES)]
            vals = plsc.load_gather(row_v, [iv])
            valid = (lanes + j * LANES) < pos
            ckey_v[pl.ds(j * LANES, LANES)] = jnp.where(
                valid, _mkey(vals), np.uint32(0)
            )
            return c

        lax.fori_loop(0, vn, mk, 0)

        def bs2(_, lohi):
            lo, hi = lohi
            mid = lo + ((hi - lo) >> 1) + np.uint32(1)

            def cnt(j, acc):
                m = ckey_v[pl.ds(j * LANES, LANES)] >= mid
                return acc + jnp.where(m, 1, 0)

            c = jnp.sum(lax.fori_loop(0, vn, cnt, zero16i))
            ok = c >= TOPK
            return (jnp.where(ok, mid, lo), jnp.where(ok, hi, mid - 1))

        tau_s, _ = lax.fori_loop(
            0, 32, bs2, (np.uint32(0), np.uint32(0xFFFFFFFF))
        )

        def cgt(j, acc):
            m = ckey_v[pl.ds(j * LANES, LANES)] > tau_s
            return acc + jnp.where(m, 1, 0)

        gt_cnt = jnp.sum(lax.fori_loop(0, vn, cgt, zero16i))
        need = TOPK - gt_cnt

        def selp(j, carry2):
            outpos, eqc = carry2
            kk = ckey_v[pl.ds(j * LANES, LANES)]
            gt = kk > tau_s
            eq = kk == tau_s
            ecs = plsc.cumsum(jnp.where(eq, 1, 0))
            take = eq & ((ecs + eqc) <= need)
            sel = gt | take
            iv = cidx_v[pl.ds(j * LANES, LANES)]
            plsc.store_compressed(selidx_v.at[pl.ds(outpos, LANES)], iv, mask=sel)
            outpos = outpos + plsc.all_reduce_population_count(sel)[0]
            eqc = eqc + ecs[15]
            return (outpos, eqc)

        lax.fori_loop(0, vn, selp, (0, 0))

        def _wait_out():
            pltpu.make_async_copy(act_row, acts.at[r], sem_o).wait()

        lax.cond(first, lambda: None, _wait_out)
        for j in range(TOPK // LANES):
            ivp = selidx_p[pl.ds(j * LANES, LANES)]
            plsc.store_scatter(act_row, [ivp], zero16f)

        for j in range(TOPK // LANES):
            iv = selidx_v[pl.ds(j * LANES, LANES)]
            vv = plsc.load_gather(row_v, [iv])
            plsc.store_scatter(act_row, [iv], vv)
        pltpu.make_async_copy(act_row, acts.at[r], sem_o).start()
        for j in range(TOPK // LANES):
            selidx_p[pl.ds(j * LANES, LANES)] = selidx_v[pl.ds(j * LANES, LANES)]

    def _start(r, row_v, sem, g_v, gsem):
        pltpu.make_async_copy(pre.at[r], row_v, sem).start()
        pltpu.make_async_copy(gmax.at[r], g_v, gsem).start()

    def _wait(r, row_v, sem, g_v, gsem):
        pltpu.make_async_copy(pre.at[r], row_v, sem).wait()
        pltpu.make_async_copy(gmax.at[r], g_v, gsem).wait()

    _start(r0, row_a, sem_a, g_a, sem_ga)
    _start(r0 + 1, row_b, sem_b, g_b, sem_gb)

    def pair(i, c):
        ra = r0 + 2 * i
        _wait(ra, row_a, sem_a, g_a, sem_ga)
        process(row_a, g_a, ra, i == 0)
        lax.cond(
            2 * i + 2 < ROWS_PER_W,
            lambda: _start(ra + 2, row_a, sem_a, g_a, sem_ga),
            lambda: None,
        )
        rb = ra + 1
        _wait(rb, row_b, sem_b, g_b, sem_gb)
        process(row_b, g_b, rb, False)
        lax.cond(
            2 * i + 3 < ROWS_PER_W,
            lambda: _start(rb + 2, row_b, sem_b, g_b, sem_gb),
            lambda: None,
        )
        return c

    lax.fori_loop(0, ROWS_PER_W // 2, pair, 0)
    pltpu.make_async_copy(act_row, acts.at[r0 + ROWS_PER_W - 1], sem_o).wait()


def _sc_select(pre, gmax):
    mesh = plsc.VectorSubcoreMesh(
        core_axis_name="c", subcore_axis_name="s"
    )
    return pl.kernel(
        _sc_body,
        out_type=jax.ShapeDtypeStruct((TOKENS, FACTORS), jnp.float32),
        mesh=mesh,
        compiler_params=pltpu.CompilerParams(needs_layout_passes=False),
        scratch_types=[
            pltpu.VMEM((FACTORS,), jnp.float32),
            pltpu.VMEM((FACTORS,), jnp.float32),
            pltpu.VMEM((NBLK,), jnp.float32),
            pltpu.VMEM((NBLK,), jnp.float32),
            pltpu.VMEM((256,), jnp.uint32),
            pltpu.VMEM((BLCAP,), jnp.int32),
            pltpu.VMEM((CAP,), jnp.int32),
            pltpu.VMEM((CAP,), jnp.uint32),
            pltpu.VMEM((TOPK,), jnp.int32),
            pltpu.VMEM((TOPK,), jnp.int32),
            pltpu.VMEM((FACTORS,), jnp.float32),
            pltpu.SemaphoreType.DMA,
            pltpu.SemaphoreType.DMA,
            pltpu.SemaphoreType.DMA,
            pltpu.SemaphoreType.DMA,
            pltpu.SemaphoreType.DMA,
        ],
    )(pre, gmax)


def kernel(x, W_enc, W_dec, b_pre):
    pre, g3 = _pre_acts(x, W_enc, b_pre)
    gmax = jnp.transpose(g3, (1, 0, 2)).reshape(TOKENS, NBLK)
    acts = _sc_select(pre, gmax)
    x_hat = _decode(acts, W_dec, b_pre)
    return (x_hat, acts)


# speedup vs baseline: 1.4052x; 1.0091x over previous
import jax
import numpy as np
import jax.numpy as jnp
from jax import lax
from jax.experimental import pallas as pl
from jax.experimental.pallas import tpu as pltpu
from jax.experimental.pallas import tpu_sc as plsc

TOKENS = 8192
D_MODEL = 768
FACTORS = 32768
TOPK = 64

TB = 512
FB = 1024


BLK = 128
NBLK = FACTORS // BLK
NF = FACTORS // FB


def _matmul_body(x_ref, w_ref, b_ref, out_ref, g_ref):
    xc = x_ref[...] - b_ref[...]
    p = jnp.dot(xc, w_ref[...], preferred_element_type=jnp.float32)
    out_ref[...] = p
    g = jnp.max(p.reshape(TB, FB // BLK, BLK), axis=2)
    g_ref[...] = g.reshape(1, TB, FB // BLK)


def _pre_acts(x, W_enc, b_pre):
    return pl.pallas_call(
        _matmul_body,
        grid=(FACTORS // FB, TOKENS // TB),
        in_specs=[
            pl.BlockSpec((TB, D_MODEL), lambda f, t: (t, 0)),
            pl.BlockSpec((D_MODEL, FB), lambda f, t: (0, f)),
            pl.BlockSpec((1, D_MODEL), lambda f, t: (0, 0)),
        ],
        out_specs=[
            pl.BlockSpec((TB, FB), lambda f, t: (t, f)),
            pl.BlockSpec((1, TB, FB // BLK), lambda f, t: (f, t, 0)),
        ],
        out_shape=[
            jax.ShapeDtypeStruct((TOKENS, FACTORS), jnp.float32),
            jax.ShapeDtypeStruct((NF, TOKENS, FB // BLK), jnp.float32),
        ],
        compiler_params=pltpu.CompilerParams(
            dimension_semantics=("arbitrary", "arbitrary"),
            vmem_limit_bytes=100 * 1024 * 1024,
        ),
    )(x, W_enc, b_pre.reshape(1, D_MODEL))


KB = 4096


def _dec_body(a_ref, w_ref, b_ref, out_ref):
    k = pl.program_id(1)

    @pl.when(k == 0)
    def _():
        out_ref[...] = jnp.broadcast_to(b_ref[...], out_ref.shape)

    out_ref[...] += jnp.dot(
        a_ref[...], w_ref[...], preferred_element_type=jnp.float32
    )


def _decode(acts, W_dec, b_pre):
    return pl.pallas_call(
        _dec_body,
        grid=(TOKENS // TB, FACTORS // KB),
        in_specs=[
            pl.BlockSpec((TB, KB), lambda t, k: (t, k)),
            pl.BlockSpec((KB, D_MODEL), lambda t, k: (k, 0)),
            pl.BlockSpec((1, D_MODEL), lambda t, k: (0, 0)),
        ],
        out_specs=pl.BlockSpec((TB, D_MODEL), lambda t, k: (t, 0)),
        out_shape=jax.ShapeDtypeStruct((TOKENS, D_MODEL), jnp.float32),
        compiler_params=pltpu.CompilerParams(
            dimension_semantics=("arbitrary", "arbitrary"),
        ),
    )(acts, W_dec, b_pre.reshape(1, D_MODEL))


SC_CORES = 2
SC_SUBCORES = 16
LANES = 16
NW = SC_CORES * SC_SUBCORES
ROWS_PER_W = TOKENS // NW
NVEC = FACTORS // LANES
GROUPS = 256
GVREGS = GROUPS // LANES
VPG = NVEC // GROUPS * LANES
CAP = 512
BLCAP = 512

_SIGN = np.uint32(0x80000000)
_MANT = np.uint32(0x7FFFFFFF)


def _mkey(v):
    b = lax.bitcast_convert_type(v, jnp.uint32)
    neg = (b >> 31) == 1
    return jnp.where(neg, ~b, b | _SIGN)


def _inv_key(u):
    is_pos = (u >> 31) == 1
    b = jnp.where(is_pos, u & _MANT, ~u)
    return lax.bitcast_convert_type(b, jnp.float32)


def _sc_body(pre, gmax, acts, row_a, row_b, g_a, g_b, gk_v, blist_v,
             cidx_v, ckey_v, selidx_v, selidx_p, act_row,
             sem_a, sem_b, sem_ga, sem_gb, sem_o):
    wid = lax.axis_index("s") * SC_CORES + lax.axis_index("c")
    r0 = wid * ROWS_PER_W
    lanes = lax.iota(jnp.int32, LANES)
    zero16f = jnp.zeros((LANES,), jnp.float32)
    zero16i = jnp.zeros((LANES,), jnp.int32)

    def _z_act(i, c):
        act_row[pl.ds(i * LANES, LANES)] = zero16f
        return c

    lax.fori_loop(0, NVEC, _z_act, 0)

    def _z_cidx(i, c):
        cidx_v[pl.ds(i * LANES, LANES)] = zero16i
        return c

    lax.fori_loop(0, CAP // LANES, _z_cidx, 0)

    def _z_blist(i, c):
        blist_v[pl.ds(i * LANES, LANES)] = zero16i
        return c

    lax.fori_loop(0, BLCAP // LANES, _z_blist, 0)
    for j in range(TOPK // LANES):
        selidx_v[pl.ds(j * LANES, LANES)] = zero16i
        selidx_p[pl.ds(j * LANES, LANES)] = zero16i

    NGV = NBLK // LANES

    def process(row_v, g_v, r, first):
        gks = [_mkey(g_v[pl.ds(b * LANES, LANES)]) for b in range(NGV)]

        def bs1(_, lohi):
            lo, hi = lohi
            mid = lo + ((hi - lo) >> 1) + np.uint32(1)
            cvec = zero16i
            for gv in gks:
                cvec = cvec + jnp.where(gv >= mid, 1, 0)
            c = jnp.sum(cvec)
            ok = c >= TOPK
            return (jnp.where(ok, mid, lo), jnp.where(ok, hi, mid - 1))

        tau_m, _ = lax.fori_loop(
            0, 20, bs1, (np.uint32(0), np.uint32(0xFFFFFFFF))
        )
        tau_f = _inv_key(tau_m)

        def blb(g8, bp):
            ms = []
            cs = []
            for k in range(8):
                g = g8 * 8 + k
                m = g_v[pl.ds(g * LANES, LANES)] >= tau_f
                ms.append((g, m))
                cs.append(plsc.all_reduce_population_count(m)[0])
            offs = [bp]
            for k in range(1, 8):
                offs.append(jnp.minimum(offs[-1] + cs[k - 1], BLCAP - LANES))
            for k, (g, m) in enumerate(ms):
                ids = lanes + g * LANES
                plsc.store_compressed(
                    blist_v.at[pl.ds(offs[k], LANES)], ids, mask=m
                )
            return jnp.minimum(offs[-1] + cs[-1], BLCAP - LANES)

        bpos = lax.fori_loop(0, NGV // 8, blb, 0)

        def ext(bj, pos):
            b = blist_v[pl.ds(bj, LANES)][0]
            mms = []
            cs = []
            for k in range(BLK // LANES):
                off = b * BLK + k * LANES
                mm = row_v[pl.ds(off, LANES)] >= tau_f
                mms.append((off, mm))
                cs.append(plsc.all_reduce_population_count(mm)[0])
            offs = [pos]
            for k in range(1, BLK // LANES):
                offs.append(jnp.minimum(offs[-1] + cs[k - 1], CAP - LANES))
            for k, (off, mm) in enumerate(mms):
                iv = lanes + off
                plsc.store_compressed(
                    cidx_v.at[pl.ds(offs[k], LANES)], iv, mask=mm
                )
            return jnp.minimum(offs[-1] + cs[-1], CAP - LANES)

        pos = lax.fori_loop(0, bpos, ext, 0)
        vn = (pos + LANES - 1) // LANES

        def mk(j, c):
            iv = cidx_v[pl.ds(j * LANES, LANES)]
            vals = plsc.load_gather(row_v, [iv])
            valid = (lanes + j * LANES) < pos
            ckey_v[pl.ds(j * LANES, LANES)] = jnp.where(
                valid, _mkey(vals), np.uint32(0)
            )
            return c

        lax.fori_loop(0, vn, mk, 0)

        def bs2(_, lohi):
            lo, hi = lohi
            mid = lo + ((hi - lo) >> 1) + np.uint32(1)

            def cnt(j, acc):
                m = ckey_v[pl.ds(j * LANES, LANES)] >= mid
                return acc + jnp.where(m, 1, 0)

            c = jnp.sum(lax.fori_loop(0, vn, cnt, zero16i))
            ok = c >= TOPK
            return (jnp.where(ok, mid, lo), jnp.where(ok, hi, mid - 1))

        tau_s, _ = lax.fori_loop(
            0, 32, bs2, (np.uint32(0), np.uint32(0xFFFFFFFF))
        )

        def cgt(j, acc):
            m = ckey_v[pl.ds(j * LANES, LANES)] > tau_s
            return acc + jnp.where(m, 1, 0)

        gt_cnt = jnp.sum(lax.fori_loop(0, vn, cgt, zero16i))
        need = TOPK - gt_cnt

        def selp(j, carry2):
            outpos, eqc = carry2
            kk = ckey_v[pl.ds(j * LANES, LANES)]
            gt = kk > tau_s
            eq = kk == tau_s
            ecs = plsc.cumsum(jnp.where(eq, 1, 0))
            take = eq & ((ecs + eqc) <= need)
            sel = gt | take
            iv = cidx_v[pl.ds(j * LANES, LANES)]
            plsc.store_compressed(selidx_v.at[pl.ds(outpos, LANES)], iv, mask=sel)
            outpos = outpos + plsc.all_reduce_population_count(sel)[0]
            eqc = eqc + ecs[15]
            return (outpos, eqc)

        lax.fori_loop(0, vn, selp, (0, 0))

        def _wait_out():
            pltpu.make_async_copy(act_row, acts.at[r], sem_o).wait()

        lax.cond(first, lambda: None, _wait_out)
        for j in range(TOPK // LANES):
            ivp = selidx_p[pl.ds(j * LANES, LANES)]
            plsc.store_scatter(act_row, [ivp], zero16f)

        for j in range(TOPK // LANES):
            iv = selidx_v[pl.ds(j * LANES, LANES)]
            vv = plsc.load_gather(row_v, [iv])
            plsc.store_scatter(act_row, [iv], vv)
        pltpu.make_async_copy(act_row, acts.at[r], sem_o).start()
        for j in range(TOPK // LANES):
            selidx_p[pl.ds(j * LANES, LANES)] = selidx_v[pl.ds(j * LANES, LANES)]

    def _start(r, row_v, sem, g_v, gsem):
        pltpu.make_async_copy(pre.at[r], row_v, sem).start()
        pltpu.make_async_copy(gmax.at[r], g_v, gsem).start()

    def _wait(r, row_v, sem, g_v, gsem):
        pltpu.make_async_copy(pre.at[r], row_v, sem).wait()
        pltpu.make_async_copy(gmax.at[r], g_v, gsem).wait()

    _start(r0, row_a, sem_a, g_a, sem_ga)
    _start(r0 + 1, row_b, sem_b, g_b, sem_gb)

    def pair(i, c):
        ra = r0 + 2 * i
        _wait(ra, row_a, sem_a, g_a, sem_ga)
        process(row_a, g_a, ra, i == 0)
        lax.cond(
            2 * i + 2 < ROWS_PER_W,
            lambda: _start(ra + 2, row_a, sem_a, g_a, sem_ga),
            lambda: None,
        )
        rb = ra + 1
        _wait(rb, row_b, sem_b, g_b, sem_gb)
        process(row_b, g_b, rb, False)
        lax.cond(
            2 * i + 3 < ROWS_PER_W,
            lambda: _start(rb + 2, row_b, sem_b, g_b, sem_gb),
            lambda: None,
        )
        return c

    lax.fori_loop(0, ROWS_PER_W // 2, pair, 0)
    pltpu.make_async_copy(act_row, acts.at[r0 + ROWS_PER_W - 1], sem_o).wait()


def _sc_select(pre, gmax):
    mesh = plsc.VectorSubcoreMesh(
        core_axis_name="c", subcore_axis_name="s"
    )
    return pl.kernel(
        _sc_body,
        out_type=jax.ShapeDtypeStruct((TOKENS, FACTORS), jnp.float32),
        mesh=mesh,
        compiler_params=pltpu.CompilerParams(needs_layout_passes=False),
        scratch_types=[
            pltpu.VMEM((FACTORS,), jnp.float32),
            pltpu.VMEM((FACTORS,), jnp.float32),
            pltpu.VMEM((NBLK,), jnp.float32),
            pltpu.VMEM((NBLK,), jnp.float32),
            pltpu.VMEM((256,), jnp.uint32),
            pltpu.VMEM((BLCAP,), jnp.int32),
            pltpu.VMEM((CAP,), jnp.int32),
            pltpu.VMEM((CAP,), jnp.uint32),
            pltpu.VMEM((TOPK,), jnp.int32),
            pltpu.VMEM((TOPK,), jnp.int32),
            pltpu.VMEM((FACTORS,), jnp.float32),
            pltpu.SemaphoreType.DMA,
            pltpu.SemaphoreType.DMA,
            pltpu.SemaphoreType.DMA,
            pltpu.SemaphoreType.DMA,
            pltpu.SemaphoreType.DMA,
        ],
    )(pre, gmax)


def kernel(x, W_enc, W_dec, b_pre):
    pre, g3 = _pre_acts(x, W_enc, b_pre)
    gmax = jnp.transpose(g3, (1, 0, 2)).reshape(TOKENS, NBLK)
    acts = _sc_select(pre, gmax)
    x_hat = _decode(acts, W_dec, b_pre)
    return (x_hat, acts)
